# baseline (speedup 1.0000x reference)
import functools

import jax
import jax.numpy as jnp
from jax.experimental import pallas as pl
from jax.experimental.pallas import tpu as pltpu

_DT = 0.1
_BF16 = jnp.bfloat16
_N_STEPS = 7


def _unet_kernel(x_ref, border_ref, enc_w_ref, bias_ref,
                 mid_w_ref, dec_w_ref, out_w_ref,
                 o_ref, h1b_ref, pooled_ref, ub_ref, acc_ref,
                 *, N, H, W, hidden, dt):
    f32 = jnp.float32
    HW = H * W
    L = N * HW
    G = hidden // 2
    step = pl.program_id(0)

    def shift(v, s):
        k = (-s) % L
        return v if k == 0 else pltpu.roll(v, k, 1)

    def tap_piece(vb, tap, dil, mask_base):
        kh, kw = tap // 3, tap % 3
        dh, dw = (kh - 1) * dil, (kw - 1) * dil
        t = shift(vb, dh * W + dw)
        if dh != 0 or dw != 0:
            r = mask_base + tap
            t = t * border_ref[r:r + 1, :].astype(_BF16)
        return t

    def interleaved_stack(src, dil, mask_base):
        pieces = [tap_piece(src, t, dil, mask_base) for t in range(9)]
        return jnp.stack(pieces, axis=1).reshape(9 * G, L)

    def bias_col(c):
        return jnp.broadcast_to(bias_ref[:, c:c + 1], (hidden, L))

    @pl.when(step == 0)
    def _enc_pool():
        xb = x_ref[...].astype(_BF16)
        pieces = [tap_piece(xb, t, 1, 0) for t in range(9)]
        pieces.append(jnp.ones((1, L), _BF16))
        stk = jnp.concatenate(pieces, axis=0)
        h1 = jnp.maximum(
            jnp.dot(enc_w_ref[...], stk, preferred_element_type=f32), 0.0)
        h1b = h1.astype(_BF16)
        h1b_ref[...] = h1b

        col = jax.lax.broadcasted_iota(jnp.int32, (1, L), 1)
        w_even = (col % 2) == 0
        h_even = ((col // W) % 2) == 0
        p_w = jnp.where(w_even,
                        jnp.maximum(h1b, shift(h1b, 1)),
                        jnp.maximum(h1b, shift(h1b, -1)))
        pooled_ref[...] = jnp.where(h_even,
                                    jnp.maximum(p_w, shift(p_w, W)),
                                    jnp.maximum(p_w, shift(p_w, -W)))
        acc_ref[...] = bias_col(1)

    for s in range(2):
        @pl.when(step == s)
        def _mid_group(s=s):
            src = pooled_ref[s * G:(s + 1) * G, :]
            stk = interleaved_stack(src, 2, 9)
            acc_ref[...] = acc_ref[...] + jnp.dot(
                mid_w_ref[...], stk, preferred_element_type=f32)

    @pl.when(step == 1)
    def _mid_done():
        ub_ref[...] = jnp.maximum(acc_ref[...], 0.0).astype(_BF16)
        acc_ref[...] = bias_col(2)

    for s in range(4):
        @pl.when(step == 2 + s)
        def _dec_group(s=s):
            src_ref = h1b_ref if s < 2 else ub_ref
            g = s % 2
            src = src_ref[g * G:(g + 1) * G, :]
            stk = interleaved_stack(src, 1, 0)
            acc_ref[...] = acc_ref[...] + jnp.dot(
                dec_w_ref[...], stk, preferred_element_type=f32)

    @pl.when(step == _N_STEPS - 1)
    def _out_mask():
        d = jnp.maximum(acc_ref[...], 0.0).astype(_BF16)
        y = jnp.dot(out_w_ref[...],
                    jnp.concatenate([d, jnp.ones((1, L), _BF16)], axis=0),
                    preferred_element_type=f32)
        x = x_ref[...]
        x0 = x[0:1, 0:HW]
        z0 = x[1:2, 0:HW]
        t1 = x[2:3, 0:HW] + dt
        m = jnp.where(x0 * x0 + z0 * z0 <= t1 * t1, 1.0, 0.0)
        if N > 1:
            m = jnp.concatenate([m] * N, axis=1)
        o_ref[...] = (y * m).astype(o_ref.dtype)


def _border_masks(N, H, W):
    L = N * H * W
    col = jnp.arange(L, dtype=jnp.int32)
    w_pos = col % W
    h_pos = (col // W) % H
    rows = []
    for dil in (1, 2):
        for kh in range(3):
            for kw in range(3):
                dh, dw = (kh - 1) * dil, (kw - 1) * dil
                valid = ((h_pos + dh >= 0) & (h_pos + dh < H) &
                         (w_pos + dw >= 0) & (w_pos + dw < W))
                rows.append(valid)
    return jnp.stack(rows, axis=0).astype(jnp.float32)


def _enc_stack(w, b, cin_pad):
    cout, cin, kh, kw = w.shape
    wt = jnp.transpose(w, (0, 2, 3, 1))
    if cin_pad != cin:
        wt = jnp.pad(wt, ((0, 0), (0, 0), (0, 0), (0, cin_pad - cin)))
    wt = wt.reshape(cout, kh * kw * cin_pad)
    return jnp.concatenate([wt, b.reshape(cout, 1)], axis=1).astype(_BF16)


def kernel(enc_w, enc_b, mid_w, mid_b, dec_w, dec_b, out_w, out_b, x):
    N, ci, H, W = x.shape
    hidden = enc_w.shape[0]
    co = out_w.shape[0]
    HW, L = H * W, N * H * W
    ci_p = max(8, ((ci + 7) // 8) * 8)
    co_p = max(8, ((co + 7) // 8) * 8)
    G = hidden // 2

    x_cl = jnp.transpose(x.reshape(N, ci, HW), (1, 0, 2)).reshape(ci, L)
    if ci_p != ci:
        x_cl = jnp.pad(x_cl, ((0, ci_p - ci), (0, 0)))

    border = _border_masks(N, H, W)

    enc_ws = _enc_stack(enc_w, enc_b, ci_p)
    mid_ws = mid_w.astype(_BF16).reshape(hidden, 9 * hidden)
    dec_ws = dec_w.astype(_BF16).reshape(hidden, 18 * hidden)
    out_ws = jnp.concatenate(
        [out_w.reshape(co, hidden), out_b.reshape(co, 1)], axis=1)
    if co_p != co:
        out_ws = jnp.pad(out_ws, ((0, co_p - co), (0, 0)))
    out_ws = out_ws.astype(_BF16)

    biases = jnp.zeros((hidden, 128), jnp.float32)
    biases = biases.at[:, 1].set(mid_b).at[:, 2].set(dec_b)

    kfn = functools.partial(_unet_kernel, N=N, H=H, W=W,
                            hidden=hidden, dt=float(_DT))

    flops = 2 * L * (hidden * (9 * ci_p + 1) + hidden * (9 * hidden + 1)
                     + hidden * (18 * hidden + 1) + co_p * (hidden + 1))
    bytes_accessed = int(4 * (x_cl.size + border.size + biases.size
                              + co_p * L)
                         + 2 * (enc_ws.size + mid_ws.size + dec_ws.size
                                + out_ws.size))

    out = pl.pallas_call(
        kfn,
        out_shape=jax.ShapeDtypeStruct((co_p, L), jnp.float32),
        grid=(_N_STEPS,),
        in_specs=[
            pl.BlockSpec((ci_p, L), lambda i: (0, 0)),
            pl.BlockSpec(border.shape, lambda i: (0, 0)),
            pl.BlockSpec(enc_ws.shape, lambda i: (0, 0)),
            pl.BlockSpec(biases.shape, lambda i: (0, 0)),
            pl.BlockSpec((hidden, 9 * G),
                         lambda i: (0, jnp.minimum(i, 1))),
            pl.BlockSpec((hidden, 9 * G),
                         lambda i: (0, jnp.clip(i - 2, 0, 3))),
            pl.BlockSpec(out_ws.shape, lambda i: (0, 0)),
        ],
        out_specs=pl.BlockSpec((co_p, L), lambda i: (0, 0)),
        scratch_shapes=[
            pltpu.VMEM((hidden, L), _BF16),
            pltpu.VMEM((hidden, L), _BF16),
            pltpu.VMEM((hidden, L), _BF16),
            pltpu.VMEM((hidden, L), jnp.float32),
        ],
        compiler_params=pltpu.CompilerParams(
            dimension_semantics=("arbitrary",)),
        cost_estimate=pl.CostEstimate(flops=flops, transcendentals=0,
                                      bytes_accessed=bytes_accessed),
    )(x_cl, border, enc_ws, biases, mid_ws, dec_ws, out_ws)

    return out.reshape(co_p, N, H, W).transpose(1, 0, 2, 3)[:, :co]


# speedup vs baseline: 1.0011x; 1.0011x over previous
import functools

import jax
import jax.numpy as jnp
from jax.experimental import pallas as pl
from jax.experimental.pallas import tpu as pltpu

_DT = 0.1
_BF16 = jnp.bfloat16
_N_STEPS = 7


def _unet_kernel(x_ref, border_ref, enc_w_ref, bias_ref,
                 mid_w_ref, dec_w_ref, out_w_ref,
                 o_ref, h1b_ref, pooled_ref, ub_ref, acc_ref,
                 *, N, H, W, hidden, dt):
    f32 = jnp.float32
    HW = H * W
    L = N * HW
    G = hidden // 2
    step = pl.program_id(0)

    def shift(v, s):
        k = (-s) % L
        return v if k == 0 else pltpu.roll(v, k, 1)

    def tap_piece(vb, tap, dil, mask_base):
        kh, kw = tap // 3, tap % 3
        dh, dw = (kh - 1) * dil, (kw - 1) * dil
        t = shift(vb, dh * W + dw)
        if dh != 0 or dw != 0:
            r = mask_base + tap
            t = t * border_ref[r:r + 1, :].astype(_BF16)
        return t

    def interleaved_stack(src, dil, mask_base):
        pieces = [tap_piece(src, t, dil, mask_base) for t in range(9)]
        return jnp.stack(pieces, axis=1).reshape(9 * G, L)

    def bias_col(c):
        return jnp.broadcast_to(bias_ref[:, c:c + 1], (hidden, L))

    @pl.when(step == 0)
    def _enc_pool():
        xb = x_ref[...].astype(_BF16)
        pieces = [tap_piece(xb, t, 1, 0) for t in range(9)]
        pieces.append(jnp.ones((1, L), _BF16))
        stk = jnp.concatenate(pieces, axis=0)
        h1 = jnp.maximum(
            jnp.dot(enc_w_ref[...], stk, preferred_element_type=f32), 0.0)
        h1b = h1.astype(_BF16)
        h1b_ref[...] = h1b

        col = jax.lax.broadcasted_iota(jnp.int32, (1, L), 1)
        w_even = (col % 2) == 0
        h_even = ((col // W) % 2) == 0
        p_w = jnp.where(w_even,
                        jnp.maximum(h1b, shift(h1b, 1)),
                        jnp.maximum(h1b, shift(h1b, -1)))
        pooled_ref[...] = jnp.where(h_even,
                                    jnp.maximum(p_w, shift(p_w, W)),
                                    jnp.maximum(p_w, shift(p_w, -W)))
        acc_ref[...] = bias_col(1)

    for s in range(2):
        @pl.when(step == s)
        def _mid_group(s=s):
            src = pooled_ref[s * G:(s + 1) * G, :]
            stk = interleaved_stack(src, 2, 9)
            acc_ref[...] = acc_ref[...] + jnp.dot(
                mid_w_ref[...], stk, preferred_element_type=f32)

    @pl.when(step == 1)
    def _mid_done():
        ub_ref[...] = jnp.maximum(acc_ref[...], 0.0).astype(_BF16)
        acc_ref[...] = bias_col(2)

    for s in range(4):
        @pl.when(step == 2 + s)
        def _dec_group(s=s):
            src_ref = h1b_ref if s < 2 else ub_ref
            g = s % 2
            src = src_ref[g * G:(g + 1) * G, :]
            stk = interleaved_stack(src, 1, 0)
            acc_ref[...] = acc_ref[...] + jnp.dot(
                dec_w_ref[...], stk, preferred_element_type=f32)

    @pl.when(step == _N_STEPS - 1)
    def _out_mask():
        d = jnp.maximum(acc_ref[...], 0.0).astype(_BF16)
        y = jnp.dot(out_w_ref[...],
                    jnp.concatenate([d, jnp.ones((1, L), _BF16)], axis=0),
                    preferred_element_type=f32)
        x = x_ref[...]
        x0 = x[0:1, 0:HW]
        z0 = x[1:2, 0:HW]
        t1 = x[2:3, 0:HW] + dt
        m = jnp.where(x0 * x0 + z0 * z0 <= t1 * t1, 1.0, 0.0)
        if N > 1:
            m = jnp.concatenate([m] * N, axis=1)
        o_ref[...] = (y * m).astype(o_ref.dtype)


def _border_masks(N, H, W):
    L = N * H * W
    col = jnp.arange(L, dtype=jnp.int32)
    w_pos = col % W
    h_pos = (col // W) % H
    rows = []
    for dil in (1, 2):
        for kh in range(3):
            for kw in range(3):
                dh, dw = (kh - 1) * dil, (kw - 1) * dil
                valid = ((h_pos + dh >= 0) & (h_pos + dh < H) &
                         (w_pos + dw >= 0) & (w_pos + dw < W))
                rows.append(valid)
    return jnp.stack(rows, axis=0).astype(jnp.float32)


def _enc_stack(w, b, cin_pad):
    cout, cin, kh, kw = w.shape
    wt = jnp.transpose(w, (0, 2, 3, 1))
    if cin_pad != cin:
        wt = jnp.pad(wt, ((0, 0), (0, 0), (0, 0), (0, cin_pad - cin)))
    wt = wt.reshape(cout, kh * kw * cin_pad)
    return jnp.concatenate([wt, b.reshape(cout, 1)], axis=1).astype(_BF16)


def kernel(enc_w, enc_b, mid_w, mid_b, dec_w, dec_b, out_w, out_b, x):
    N, ci, H, W = x.shape
    hidden = enc_w.shape[0]
    co = out_w.shape[0]
    HW, L = H * W, N * H * W
    ci_p = max(8, ((ci + 7) // 8) * 8)
    co_p = max(8, ((co + 7) // 8) * 8)
    G = hidden // 2

    x_cl = jnp.transpose(x.reshape(N, ci, HW), (1, 0, 2)).reshape(ci, L)
    if ci_p != ci:
        x_cl = jnp.pad(x_cl, ((0, ci_p - ci), (0, 0)))

    border = _border_masks(N, H, W)

    enc_ws = _enc_stack(enc_w, enc_b, ci_p)
    mid_ws = mid_w.reshape(hidden, 9 * hidden).astype(_BF16)
    dec_ws = dec_w.reshape(hidden, 18 * hidden).astype(_BF16)
    out_ws = jnp.concatenate(
        [out_w.reshape(co, hidden), out_b.reshape(co, 1)], axis=1)
    if co_p != co:
        out_ws = jnp.pad(out_ws, ((0, co_p - co), (0, 0)))
    out_ws = out_ws.astype(_BF16)

    biases = jnp.zeros((hidden, 128), jnp.float32)
    biases = biases.at[:, 1].set(mid_b).at[:, 2].set(dec_b)

    kfn = functools.partial(_unet_kernel, N=N, H=H, W=W,
                            hidden=hidden, dt=float(_DT))

    flops = 2 * L * (hidden * (9 * ci_p + 1) + hidden * (9 * hidden + 1)
                     + hidden * (18 * hidden + 1) + co_p * (hidden + 1))
    bytes_accessed = int(4 * (x_cl.size + border.size + biases.size
                              + co_p * L)
                         + 2 * (enc_ws.size + mid_ws.size + dec_ws.size
                                + out_ws.size))

    out = pl.pallas_call(
        kfn,
        out_shape=jax.ShapeDtypeStruct((co_p, L), jnp.float32),
        grid=(_N_STEPS,),
        in_specs=[
            pl.BlockSpec((ci_p, L), lambda i: (0, 0)),
            pl.BlockSpec(border.shape, lambda i: (0, 0)),
            pl.BlockSpec(enc_ws.shape, lambda i: (0, 0)),
            pl.BlockSpec(biases.shape, lambda i: (0, 0)),
            pl.BlockSpec((hidden, 9 * G),
                         lambda i: (0, jnp.minimum(i, 1))),
            pl.BlockSpec((hidden, 9 * G),
                         lambda i: (0, jnp.clip(i - 2, 0, 3))),
            pl.BlockSpec(out_ws.shape, lambda i: (0, 0)),
        ],
        out_specs=pl.BlockSpec((co_p, L), lambda i: (0, 0)),
        scratch_shapes=[
            pltpu.VMEM((hidden, L), _BF16),
            pltpu.VMEM((hidden, L), _BF16),
            pltpu.VMEM((hidden, L), _BF16),
            pltpu.VMEM((hidden, L), jnp.float32),
        ],
        compiler_params=pltpu.CompilerParams(
            dimension_semantics=("arbitrary",)),
        cost_estimate=pl.CostEstimate(flops=flops, transcendentals=0,
                                      bytes_accessed=bytes_accessed),
    )(x_cl, border, enc_ws, biases, mid_ws, dec_ws, out_ws)

    return out.reshape(co_p, N, H, W).transpose(1, 0, 2, 3)[:, :co]


# speedup vs baseline: 1.9066x; 1.9045x over previous
import functools

import jax
import jax.numpy as jnp
from jax.experimental import pallas as pl
from jax.experimental.pallas import tpu as pltpu

_DT = 0.1
_BF16 = jnp.bfloat16


def _shift(v, s, L):
    k = (-s) % L
    return v if k == 0 else pltpu.roll(v, k, 1)


def _tap_piece(vb, border_ref, tap, dil, mask_base, W, L):
    kh, kw = tap // 3, tap % 3
    dh, dw = (kh - 1) * dil, (kw - 1) * dil
    t = _shift(vb, dh * W + dw, L)
    if dh != 0 or dw != 0:
        t = t * border_ref[mask_base + tap:mask_base + tap + 1, :].astype(_BF16)
    return t


def _enc_pool(x_ref, border_ref, enc_w_ref, W, L):
    xb = x_ref[...].astype(_BF16)
    pieces = [_tap_piece(xb, border_ref, t, 1, 0, W, L) for t in range(9)]
    pieces.append(jnp.ones((1, L), _BF16))
    stk = jnp.concatenate(pieces, axis=0)
    h1 = jnp.maximum(
        jnp.dot(enc_w_ref[...], stk, preferred_element_type=jnp.float32), 0.0)
    h1b = h1.astype(_BF16)
    col = jax.lax.broadcasted_iota(jnp.int32, (1, L), 1)
    w_even = (col % 2) == 0
    h_even = ((col // W) % 2) == 0
    p_w = jnp.where(w_even,
                    jnp.maximum(h1b, _shift(h1b, 1, L)),
                    jnp.maximum(h1b, _shift(h1b, -1, L)))
    pooled = jnp.where(h_even,
                       jnp.maximum(p_w, _shift(p_w, W, L)),
                       jnp.maximum(p_w, _shift(p_w, -W, L)))
    return h1b, pooled


def _call_a_kernel(x_ref, border_ref, enc_w_ref, bias_ref, mid_w_ref,
                   h1b_out_ref, u_out_ref, pooled_ref, acc_ref,
                   *, W, L, hidden, mh):
    c = pl.program_id(0)
    s = pl.program_id(1)

    @pl.when(s == 0)
    def _():
        h1b, pooled = _enc_pool(x_ref, border_ref, enc_w_ref, W, L)
        pooled_ref[...] = pooled
        acc_ref[...] = jnp.broadcast_to(bias_ref[:, 1:2], (mh, L))

        @pl.when(c == 0)
        def _():
            h1b_out_ref[...] = h1b[:mh]

        @pl.when(c == 1)
        def _():
            h1b_out_ref[...] = h1b[mh:]

    pooled = pooled_ref[...]
    acc = acc_ref[...]
    for sv in range(3):
        @pl.when(s == sv)
        def _(sv=sv):
            a = acc
            for j in range(3):
                tap = 3 * sv + j
                piece = _tap_piece(pooled, border_ref, tap, 2, 9, W, L)
                a2 = jnp.dot(mid_w_ref[:, j * hidden:(j + 1) * hidden],
                             piece, preferred_element_type=jnp.float32)
                a = a + a2
            acc_ref[...] = a

    @pl.when(s == 2)
    def _():
        u_out_ref[...] = jnp.maximum(acc_ref[...], 0.0).astype(_BF16)


def _call_b_kernel(border_ref, bias_ref, h1b_ref, u_ref, dec_w_ref,
                   d_out_ref, acc_ref, *, W, L, hidden, mh):
    c = pl.program_id(0)
    s = pl.program_id(1)

    @pl.when(s == 0)
    def _():
        acc_ref[...] = jnp.broadcast_to(bias_ref[:, 2:3], (mh, L))

    h1b = h1b_ref[...]
    ub = u_ref[...]
    for sv in range(3):
        @pl.when(s == sv)
        def _(sv=sv):
            a = acc_ref[...]
            for j in range(3):
                tap = 3 * sv + j
                pc_h1 = _tap_piece(h1b, border_ref, tap, 1, 0, W, L)
                pc_u = _tap_piece(ub, border_ref, tap, 1, 0, W, L)
                lo = 2 * j * hidden
                a = (a
                     + jnp.dot(dec_w_ref[:, lo:lo + hidden], pc_h1,
                               preferred_element_type=jnp.float32)
                     + jnp.dot(dec_w_ref[:, lo + hidden:lo + 2 * hidden],
                               pc_u, preferred_element_type=jnp.float32))
            acc_ref[...] = a

    @pl.when(s == 2)
    def _():
        d_out_ref[...] = jnp.maximum(acc_ref[...], 0.0).astype(_BF16)


def _call_c_kernel(x_ref, d_ref, out_w_ref, o_ref, *, N, H, W, dt):
    HW = H * W
    L = N * HW
    y = jnp.dot(out_w_ref[...],
                jnp.concatenate([d_ref[...], jnp.ones((1, L), _BF16)], axis=0),
                preferred_element_type=jnp.float32)
    x = x_ref[...]
    x0 = x[0:1, 0:HW]
    z0 = x[1:2, 0:HW]
    t1 = x[2:3, 0:HW] + dt
    m = jnp.where(x0 * x0 + z0 * z0 <= t1 * t1, 1.0, 0.0)
    if N > 1:
        m = jnp.concatenate([m] * N, axis=1)
    o_ref[...] = (y * m).astype(o_ref.dtype)


def _border_masks(N, H, W):
    L = N * H * W
    col = jnp.arange(L, dtype=jnp.int32)
    w_pos = col % W
    h_pos = (col // W) % H
    rows = []
    for dil in (1, 2):
        for kh in range(3):
            for kw in range(3):
                dh, dw = (kh - 1) * dil, (kw - 1) * dil
                valid = ((h_pos + dh >= 0) & (h_pos + dh < H) &
                         (w_pos + dw >= 0) & (w_pos + dw < W))
                rows.append(valid)
    return jnp.stack(rows, axis=0).astype(jnp.float32)


def _taps_only(w, cin_pad=None):
    cout, cin, kh, kw = w.shape
    wt = jnp.transpose(w, (0, 2, 3, 1))
    if cin_pad is not None and cin_pad != cin:
        wt = jnp.pad(wt, ((0, 0), (0, 0), (0, 0), (0, cin_pad - cin)))
        cin = cin_pad
    return wt.reshape(cout, kh * kw * cin).astype(_BF16)


def kernel(enc_w, enc_b, mid_w, mid_b, dec_w, dec_b, out_w, out_b, x):
    N, ci, H, W = x.shape
    hidden = enc_w.shape[0]
    co = out_w.shape[0]
    HW, L = H * W, N * H * W
    ci_p = max(8, ((ci + 7) // 8) * 8)
    co_p = max(8, ((co + 7) // 8) * 8)
    mh = hidden // 2

    x_cl = jnp.transpose(x.reshape(N, ci, HW), (1, 0, 2)).reshape(ci, L)
    if ci_p != ci:
        x_cl = jnp.pad(x_cl, ((0, ci_p - ci), (0, 0)))

    border = _border_masks(N, H, W)

    enc_ws = jnp.concatenate(
        [_taps_only(enc_w, ci_p), enc_b.reshape(hidden, 1).astype(_BF16)],
        axis=1)
    mid_ws = _taps_only(mid_w)
    dec_ws = _taps_only(dec_w)
    out_ws = jnp.concatenate(
        [out_w.reshape(co, hidden), out_b.reshape(co, 1)], axis=1)
    if co_p != co:
        out_ws = jnp.pad(out_ws, ((0, co_p - co), (0, 0)))
    out_ws = out_ws.astype(_BF16)

    biases = jnp.zeros((hidden, 128), jnp.float32)
    biases = biases.at[:, 1].set(mid_b).at[:, 2].set(dec_b)

    a_fn = functools.partial(_call_a_kernel, W=W, L=L, hidden=hidden, mh=mh)
    h1b, u = pl.pallas_call(
        a_fn,
        out_shape=(jax.ShapeDtypeStruct((hidden, L), _BF16),
                   jax.ShapeDtypeStruct((hidden, L), _BF16)),
        grid=(2, 3),
        in_specs=[
            pl.BlockSpec((ci_p, L), lambda c, s: (0, 0)),
            pl.BlockSpec(border.shape, lambda c, s: (0, 0)),
            pl.BlockSpec(enc_ws.shape, lambda c, s: (0, 0)),
            pl.BlockSpec((mh, 128), lambda c, s: (c, 0)),
            pl.BlockSpec((mh, 3 * hidden), lambda c, s: (c, s)),
        ],
        out_specs=(pl.BlockSpec((mh, L), lambda c, s: (c, 0)),
                   pl.BlockSpec((mh, L), lambda c, s: (c, 0))),
        scratch_shapes=[
            pltpu.VMEM((hidden, L), _BF16),
            pltpu.VMEM((mh, L), jnp.float32),
        ],
        compiler_params=pltpu.CompilerParams(
            dimension_semantics=("parallel", "arbitrary")),
        cost_estimate=pl.CostEstimate(
            flops=2 * L * hidden * (9 * ci_p + 1 + 9 * hidden + 1),
            transcendentals=0,
            bytes_accessed=int(2 * mid_ws.size + 4 * x_cl.size
                               + 4 * hidden * L)),
    )(x_cl, border, enc_ws, biases, mid_ws)

    b_fn = functools.partial(_call_b_kernel, W=W, L=L, hidden=hidden, mh=mh)
    d = pl.pallas_call(
        b_fn,
        out_shape=jax.ShapeDtypeStruct((hidden, L), _BF16),
        grid=(2, 3),
        in_specs=[
            pl.BlockSpec(border.shape, lambda c, s: (0, 0)),
            pl.BlockSpec((mh, 128), lambda c, s: (c, 0)),
            pl.BlockSpec((hidden, L), lambda c, s: (0, 0)),
            pl.BlockSpec((hidden, L), lambda c, s: (0, 0)),
            pl.BlockSpec((mh, 6 * hidden), lambda c, s: (c, s)),
        ],
        out_specs=pl.BlockSpec((mh, L), lambda c, s: (c, 0)),
        scratch_shapes=[
            pltpu.VMEM((mh, L), jnp.float32),
        ],
        compiler_params=pltpu.CompilerParams(
            dimension_semantics=("parallel", "arbitrary")),
        cost_estimate=pl.CostEstimate(
            flops=2 * L * hidden * (18 * hidden + 1),
            transcendentals=0,
            bytes_accessed=int(2 * dec_ws.size + 6 * hidden * L)),
    )(border, biases, h1b, u, dec_ws)

    c_fn = functools.partial(_call_c_kernel, N=N, H=H, W=W, dt=float(_DT))
    out = pl.pallas_call(
        c_fn,
        out_shape=jax.ShapeDtypeStruct((co_p, L), jnp.float32),
        grid=(1,),
        in_specs=[
            pl.BlockSpec((ci_p, L), lambda i: (0, 0)),
            pl.BlockSpec((hidden, L), lambda i: (0, 0)),
            pl.BlockSpec(out_ws.shape, lambda i: (0, 0)),
        ],
        out_specs=pl.BlockSpec((co_p, L), lambda i: (0, 0)),
        compiler_params=pltpu.CompilerParams(
            dimension_semantics=("arbitrary",)),
        cost_estimate=pl.CostEstimate(
            flops=2 * L * co_p * (hidden + 1), transcendentals=0,
            bytes_accessed=int(2 * hidden * L + 4 * co_p * L)),
    )(x_cl, d, out_ws)

    return out.reshape(co_p, N, H, W).transpose(1, 0, 2, 3)[:, :co]


# speedup vs baseline: 2.6358x; 1.3825x over previous
import functools

import jax
import jax.numpy as jnp
from jax.experimental import pallas as pl
from jax.experimental.pallas import tpu as pltpu

_DT = 0.1
_BF16 = jnp.bfloat16

_N_STEPS = 7


def _unet_kernel(x_ref, border_ref, enc_w_ref, bias_ref,
                 mid_w_ref, dec_w_ref, out_w_ref,
                 o_ref, h1b_ref, pooled_ref, ub_ref, acc_ref,
                 *, N, H, W, hidden, dt):
    f32 = jnp.float32
    HW = H * W
    L = N * HW
    step = pl.program_id(0)

    def shift(v, s):
        k = (-s) % L
        return v if k == 0 else pltpu.roll(v, k, 1)

    def tap_piece(vb, tap, dil, mask_base):
        kh, kw = tap // 3, tap % 3
        dh, dw = (kh - 1) * dil, (kw - 1) * dil
        t = shift(vb, dh * W + dw)
        if dh != 0 or dw != 0:
            r = mask_base + tap
            t = t * border_ref[r:r + 1, :].astype(_BF16)
        return t

    def bias_col(c):
        return jnp.broadcast_to(bias_ref[:, c:c + 1], (hidden, L))

    @pl.when(step == 0)
    def _enc_pool():
        xb = x_ref[...].astype(_BF16)
        pieces = [tap_piece(xb, t, 1, 0) for t in range(9)]
        pieces.append(jnp.ones((1, L), _BF16))
        stk = jnp.concatenate(pieces, axis=0)
        h1 = jnp.maximum(
            jnp.dot(enc_w_ref[...], stk, preferred_element_type=f32), 0.0)
        h1b = h1.astype(_BF16)
        h1b_ref[...] = h1b

        col = jax.lax.broadcasted_iota(jnp.int32, (1, L), 1)
        w_even = (col % 2) == 0
        h_even = ((col // W) % 2) == 0
        p_w = jnp.where(w_even,
                        jnp.maximum(h1b, shift(h1b, 1)),
                        jnp.maximum(h1b, shift(h1b, -1)))
        pooled_ref[...] = jnp.where(h_even,
                                    jnp.maximum(p_w, shift(p_w, W)),
                                    jnp.maximum(p_w, shift(p_w, -W)))
        acc_ref[...] = bias_col(1)

    for s in range(3):
        @pl.when(step == s)
        def _mid_block(s=s):
            pooled = pooled_ref[...]
            acc = acc_ref[...]
            for j in range(3):
                tap = 3 * s + j
                pc = tap_piece(pooled, tap, 2, 9)
                acc = acc + jnp.dot(mid_w_ref[:, j * hidden:(j + 1) * hidden],
                                    pc, preferred_element_type=f32)
            acc_ref[...] = acc

    @pl.when(step == 2)
    def _mid_done():
        ub_ref[...] = jnp.maximum(acc_ref[...], 0.0).astype(_BF16)
        acc_ref[...] = bias_col(2)

    for s in range(3):
        @pl.when(step == 3 + s)
        def _dec_taps(s=s):
            h1b = h1b_ref[...]
            ub = ub_ref[...]
            acc = acc_ref[...]
            for j in range(3):
                tap = 3 * s + j
                pc_h1 = tap_piece(h1b, tap, 1, 0)
                pc_u = tap_piece(ub, tap, 1, 0)
                lo = 2 * j * hidden
                acc = (acc
                       + jnp.dot(dec_w_ref[:, lo:lo + hidden], pc_h1,
                                 preferred_element_type=f32)
                       + jnp.dot(dec_w_ref[:, lo + hidden:lo + 2 * hidden],
                                 pc_u, preferred_element_type=f32))
            acc_ref[...] = acc

    @pl.when(step == _N_STEPS - 1)
    def _out_mask():
        d = jnp.maximum(acc_ref[...], 0.0).astype(_BF16)
        y = jnp.dot(out_w_ref[...],
                    jnp.concatenate([d, jnp.ones((1, L), _BF16)], axis=0),
                    preferred_element_type=f32)
        x = x_ref[...]
        x0 = x[0:1, 0:HW]
        z0 = x[1:2, 0:HW]
        t1 = x[2:3, 0:HW] + dt
        m = jnp.where(x0 * x0 + z0 * z0 <= t1 * t1, 1.0, 0.0)
        if N > 1:
            m = jnp.concatenate([m] * N, axis=1)
        o_ref[...] = (y * m).astype(o_ref.dtype)


def _border_masks(N, H, W):
    L = N * H * W
    col = jnp.arange(L, dtype=jnp.int32)
    w_pos = col % W
    h_pos = (col // W) % H
    rows = []
    for dil in (1, 2):
        for kh in range(3):
            for kw in range(3):
                dh, dw = (kh - 1) * dil, (kw - 1) * dil
                valid = ((h_pos + dh >= 0) & (h_pos + dh < H) &
                         (w_pos + dw >= 0) & (w_pos + dw < W))
                rows.append(valid)
    return jnp.stack(rows, axis=0).astype(jnp.float32)


def _taps_only(w, cin_pad=None, dtype=_BF16):
    cout, cin, kh, kw = w.shape
    taps = [w[:, :, i, j] for i in range(kh) for j in range(kw)]
    if cin_pad is not None and cin_pad != cin:
        taps = [jnp.pad(t, ((0, 0), (0, cin_pad - cin))) for t in taps]
    return jnp.concatenate(taps, axis=1).astype(dtype)


def kernel(enc_w, enc_b, mid_w, mid_b, dec_w, dec_b, out_w, out_b, x):
    N, ci, H, W = x.shape
    hidden = enc_w.shape[0]
    co = out_w.shape[0]
    HW, L = H * W, N * H * W
    ci_p = max(8, ((ci + 7) // 8) * 8)
    co_p = max(8, ((co + 7) // 8) * 8)

    x_cl = jnp.transpose(x.reshape(N, ci, HW), (1, 0, 2)).reshape(ci, L)
    if ci_p != ci:
        x_cl = jnp.pad(x_cl, ((0, ci_p - ci), (0, 0)))

    border = _border_masks(N, H, W)

    enc_ws = jnp.concatenate(
        [_taps_only(enc_w, ci_p), enc_b.reshape(hidden, 1).astype(_BF16)],
        axis=1)
    mid_ws = _taps_only(mid_w)
    dec_ws = _taps_only(dec_w)
    out_ws = jnp.concatenate(
        [out_w.reshape(co, hidden), out_b.reshape(co, 1)], axis=1)
    if co_p != co:
        out_ws = jnp.pad(out_ws, ((0, co_p - co), (0, 0)))
    out_ws = out_ws.astype(_BF16)

    biases = jnp.zeros((hidden, 128), jnp.float32)
    biases = biases.at[:, 1].set(mid_b).at[:, 2].set(dec_b)

    kfn = functools.partial(_unet_kernel, N=N, H=H, W=W,
                            hidden=hidden, dt=float(_DT))

    flops = 2 * L * (hidden * (9 * ci_p + 1) + hidden * (9 * hidden + 1)
                     + hidden * (18 * hidden + 1) + co_p * (hidden + 1))
    bytes_accessed = int(4 * (x_cl.size + border.size + biases.size
                              + co_p * L)
                         + 2 * (enc_ws.size + mid_ws.size + dec_ws.size
                                + out_ws.size))

    out = pl.pallas_call(
        kfn,
        out_shape=jax.ShapeDtypeStruct((co_p, L), jnp.float32),
        grid=(_N_STEPS,),
        in_specs=[
            pl.BlockSpec((ci_p, L), lambda i: (0, 0)),
            pl.BlockSpec(border.shape, lambda i: (0, 0)),
            pl.BlockSpec(enc_ws.shape, lambda i: (0, 0)),
            pl.BlockSpec(biases.shape, lambda i: (0, 0)),
            pl.BlockSpec((hidden, 3 * hidden),
                         lambda i: (0, jnp.minimum(i, 2))),
            pl.BlockSpec((hidden, 6 * hidden),
                         lambda i: (0, jnp.clip(i - 3, 0, 2))),
            pl.BlockSpec(out_ws.shape, lambda i: (0, 0)),
        ],
        out_specs=pl.BlockSpec((co_p, L), lambda i: (0, 0)),
        scratch_shapes=[
            pltpu.VMEM((hidden, L), _BF16),
            pltpu.VMEM((hidden, L), _BF16),
            pltpu.VMEM((hidden, L), _BF16),
            pltpu.VMEM((hidden, L), jnp.float32),
        ],
        compiler_params=pltpu.CompilerParams(
            dimension_semantics=("arbitrary",)),
        cost_estimate=pl.CostEstimate(flops=flops, transcendentals=0,
                                      bytes_accessed=bytes_accessed),
    )(x_cl, border, enc_ws, biases, mid_ws, dec_ws, out_ws)

    return out.reshape(co_p, N, H, W).transpose(1, 0, 2, 3)[:, :co]
